# baseline (speedup 1.0000x reference)
import functools
import math

import jax
import jax.numpy as jnp
import numpy as np
from jax import lax
from jax.experimental import pallas as pl
from jax.experimental.pallas import tpu as pltpu
from jax.experimental.pallas import tpu_sc as plsc

_PAD = 0
_V = 1000
_EPS = np.float32(0.1 / 998.0)
_CONF = np.float32(0.9)
_CROW = np.float32(998.0 * float(_EPS) * math.log(float(_EPS))
                   + 0.9 * math.log(0.9))

_NC, _NS, _L = 2, 16, 16
_NW = _NC * _NS
_CR = 16
_NVF = _V // _L
_TAIL = _V - _NVF * _L


def _sum_row(buf, par, row, lane, t16, carry):
    tot_s, tot_g = carry
    zero16 = jnp.zeros((_L,), jnp.float32)
    a0 = jnp.where(lane == 0, np.float32(0.0), buf[par, row, pl.ds(0, _L)])
    a1 = buf[par, row, pl.ds(_L, _L)]
    a2 = buf[par, row, pl.ds(2 * _L, _L)]
    a3 = buf[par, row, pl.ds(3 * _L, _L)]
    k = 4
    while k + 4 <= _NVF:
        a0 = a0 + buf[par, row, pl.ds(k * _L, _L)]
        a1 = a1 + buf[par, row, pl.ds((k + 1) * _L, _L)]
        a2 = a2 + buf[par, row, pl.ds((k + 2) * _L, _L)]
        a3 = a3 + buf[par, row, pl.ds((k + 3) * _L, _L)]
        k += 4
    while k < _NVF:
        a0 = a0 + buf[par, row, pl.ds(k * _L, _L)]
        k += 1
    tail = buf[par, row, pl.ds(_V - _L, _L)]
    a1 = a1 + jnp.where(lane >= _L - _TAIL, tail, zero16)
    acc = (a0 + a1) + (a2 + a3)
    live = t16[row] != _PAD
    tot_s = tot_s + jnp.where(live, acc, zero16)
    return tot_s, tot_g


def _sc_body(x_hbm, tgt_hbm, out_hbm, buf, tvec, accv, sems):
    rps = tvec.shape[0]
    nchunk = rps // _CR
    wid = lax.axis_index("s") * _NC + lax.axis_index("c")
    base = pl.multiple_of(wid * rps, 8)
    pltpu.sync_copy(tgt_hbm.at[pl.ds(base, rps)], tvec)
    lane = lax.iota(jnp.int32, _L)

    def start(ch, par):
        for i in range(_CR):
            pltpu.async_copy(x_hbm.at[base + ch * _CR + i, :],
                             buf.at[par, i], sems.at[par])

    def wait(par):
        pltpu.make_async_copy(x_hbm.at[pl.ds(0, _CR), :], buf.at[par],
                              sems.at[par]).wait()

    start(0, 0)
    start(1, 1)

    def pair_body(c2, carry):
        tot_s, tot_g, tot_n = carry
        for sub in range(2):
            ch = 2 * c2 + sub
            t16 = tvec[pl.ds(ch * _CR, _L)]
            wait(sub)
            carry2 = (tot_s, tot_g)
            for r in range(_CR):
                carry2 = _sum_row(buf, sub, r, lane, t16, carry2)
            tot_s, tot_g = carry2
            live16 = t16 != _PAD
            parv = jnp.full((_L,), sub, jnp.int32)
            g16 = plsc.load_gather(buf, [parv, lane, t16])
            tot_g = tot_g + jnp.where(live16, g16,
                                      jnp.zeros((_L,), jnp.float32))
            tot_n = tot_n + jnp.where(live16, jnp.float32(1.0),
                                      np.float32(0.0))

            @pl.when(ch + 2 < nchunk)
            def _():
                start(ch + 2, sub)
        return tot_s, tot_g, tot_n

    z = jnp.zeros((_L,), jnp.float32)
    tot_s, tot_g, tot_n = lax.fori_loop(0, nchunk // 2, pair_body, (z, z, z))
    accv[...] = tot_n * _CROW - _EPS * tot_s - (_CONF - _EPS) * tot_g
    pltpu.sync_copy(accv, out_hbm.at[wid])


def _make_sc_loss(rows):
    rps = rows // _NW
    mesh = plsc.VectorSubcoreMesh(core_axis_name="c", subcore_axis_name="s")
    return pl.kernel(
        _sc_body,
        out_type=jax.ShapeDtypeStruct((_NW, _L), jnp.float32),
        mesh=mesh,
        compiler_params=pltpu.CompilerParams(needs_layout_passes=False),
        scratch_types=[
            pltpu.VMEM((2, _CR, _V), jnp.float32),
            pltpu.VMEM((rps,), jnp.int32),
            pltpu.VMEM((_L,), jnp.float32),
            pltpu.SemaphoreType.DMA((2,)),
        ],
    )


@jax.jit
def kernel(x, target):
    B, V = x.shape
    t32 = target.astype(jnp.int32)
    parts = _make_sc_loss(B)(x, t32)
    return jnp.sum(parts)
